# baseline (speedup 1.0000x reference)
import functools
import math

import jax
import jax.numpy as jnp
from jax import lax
from jax.experimental import pallas as pl
from jax.experimental.pallas import tpu as pltpu
from jax.experimental.pallas import tpu_sc as plsc

_V = 32000
_PAD = 0
_SMOOTHING = 0.1
_CONF = 1.0 - _SMOOTHING
_EPS = _SMOOTHING / (_V - 1)
_CS = _CONF - _EPS
_C0 = (_V - 1) * _EPS * math.log(_EPS) + _CONF * math.log(_CONF)

_R = 256
_C = 16000

_L = 16
_NC = 2
_NS = 16
_NW = _NC * _NS


def _sc_call(xrows, t, n):
    bpw = n // _NW
    vpw = _V // 128
    mesh = plsc.VectorSubcoreMesh(core_axis_name="c", subcore_axis_name="s")

    @functools.partial(
        pl.kernel, mesh=mesh,
        out_type=jax.ShapeDtypeStruct((n, 128), jnp.float32),
        scratch_types=[
            pltpu.VMEM((bpw,), jnp.int32),
            pltpu.VMEM((bpw,), jnp.int32),
            pltpu.VMEM((bpw, 128), jnp.float32),
            pltpu.SemaphoreType.DMA,
        ],
    )
    def k(x_hbm, t_hbm, out_hbm, t_v, idx_v, rows_v, sem):
        wid = lax.axis_index("s") * _NC + lax.axis_index("c")
        base = wid * bpw
        pltpu.sync_copy(t_hbm.at[pl.ds(base, bpw)], t_v)
        lanes = lax.iota(jnp.int32, _L)
        for kk in range(bpw // _L):
            tv = t_v[pl.ds(kk * _L, _L)]
            rows = (base + kk * _L + lanes) * vpw + jnp.right_shift(tv, 7)
            idx_v[pl.ds(kk * _L, _L)] = rows
        pltpu.async_copy(x_hbm.at[idx_v], rows_v, sem).wait()
        pltpu.sync_copy(rows_v, out_hbm.at[pl.ds(base, bpw)])

    return k(xrows, t)


def _tc_body(nc, inv_denom, x_ref, t_ref, g_ref, out_ref, m_ref, s_ref,
             sx_ref):
    i = pl.program_id(0)
    j = pl.program_id(1)
    xb = x_ref[...]
    t = t_ref[...]

    bmax = jnp.max(xb, axis=1, keepdims=True)
    bsum = jnp.sum(xb, axis=1, keepdims=True)

    first = j == 0
    neg_inf = jnp.full((_R, 1), -jnp.inf, dtype=jnp.float32)
    zeros = jnp.zeros((_R, 1), dtype=jnp.float32)
    m_old = jnp.where(first, neg_inf, m_ref[...])
    s_old = jnp.where(first, zeros, s_ref[...])
    sx_old = jnp.where(first, zeros, sx_ref[...])

    m_new = jnp.maximum(m_old, bmax)
    s_new = s_old * jnp.exp(m_old - m_new) + jnp.sum(
        jnp.exp(xb - m_new), axis=1, keepdims=True)
    m_ref[...] = m_new
    s_ref[...] = s_new
    sx_ref[...] = sx_old + bsum

    @pl.when(j == nc - 1)
    def _():
        g = g_ref[...]
        li = jax.lax.broadcasted_iota(jnp.int32, (_R, 128), 1)
        lane_t = jnp.bitwise_and(t, 127)
        xt = jnp.sum(jnp.where(li == lane_t, g, 0.0), axis=1, keepdims=True)
        lse = m_new + jnp.log(s_new)
        row_loss = _C0 + lse - _EPS * sx_ref[...] - _CS * xt
        valid = t != _PAD
        contrib = jnp.sum(jnp.where(valid, row_loss, 0.0)) * inv_denom
        prev = jnp.where(i == 0, jnp.zeros((1, 1), jnp.float32), out_ref[...])
        out_ref[...] = prev + contrib


def _tc_call(xf, t, g, batch):
    n = xf.shape[0]
    nr = n // _R
    nc = _V // _C
    out = pl.pallas_call(
        functools.partial(_tc_body, nc, 1.0 / batch),
        grid=(nr, nc),
        in_specs=[
            pl.BlockSpec((_R, _C), lambda i, j: (i, j)),
            pl.BlockSpec((_R, 1), lambda i, j: (i, 0)),
            pl.BlockSpec((_R, 128), lambda i, j: (i, 0)),
        ],
        out_specs=pl.BlockSpec((1, 1), lambda i, j: (0, 0)),
        out_shape=jax.ShapeDtypeStruct((1, 1), jnp.float32),
        scratch_shapes=[
            pltpu.VMEM((_R, 1), jnp.float32),
            pltpu.VMEM((_R, 1), jnp.float32),
            pltpu.VMEM((_R, 1), jnp.float32),
        ],
        compiler_params=pltpu.CompilerParams(
            dimension_semantics=("arbitrary", "arbitrary"),
        ),
    )(xf, t, g)
    return out[0, 0]


def kernel(x, target):
    batch = x.shape[0]
    n = x.shape[0] * x.shape[1]
    xf = x.reshape(n, _V)
    t = target.reshape(n).astype(jnp.int32)
    g = _sc_call(xf.reshape(n * _V // 128, 128), t, n)
    return _tc_call(xf, t.reshape(n, 1), g, batch)


# speedup vs baseline: 2.7984x; 2.7984x over previous
import functools
import math

import jax
import jax.numpy as jnp
from jax.experimental import pallas as pl
from jax.experimental.pallas import tpu as pltpu

_V = 32000
_PAD = 0
_SMOOTHING = 0.1
_CONF = 1.0 - _SMOOTHING
_EPS = _SMOOTHING / (_V - 1)
_CS = _CONF - _EPS
_C0 = (_V - 1) * _EPS * math.log(_EPS) + _CONF * math.log(_CONF)

_R = 256
_C = 16000


def _body(nc, inv_denom, x_ref, t_ref, out_ref, m_ref, s_ref, sx_ref, xt_ref):
    j = pl.program_id(1)
    xb = x_ref[...]
    t = t_ref[...]

    bmax = jnp.max(xb, axis=1, keepdims=True)
    bsum = jnp.sum(xb, axis=1, keepdims=True)

    ids = j * _C + jax.lax.broadcasted_iota(jnp.int32, (_R, _C), 1)
    hit = ids == t
    xt_part = jnp.sum(jnp.where(hit, xb, 0.0), axis=1, keepdims=True)

    first = j == 0
    neg_inf = jnp.full((_R, 1), -jnp.inf, dtype=jnp.float32)
    zeros = jnp.zeros((_R, 1), dtype=jnp.float32)
    m_old = jnp.where(first, neg_inf, m_ref[...])
    s_old = jnp.where(first, zeros, s_ref[...])
    sx_old = jnp.where(first, zeros, sx_ref[...])
    xt_old = jnp.where(first, zeros, xt_ref[...])

    m_new = jnp.maximum(m_old, bmax)
    s_new = s_old * jnp.exp(m_old - m_new) + jnp.sum(
        jnp.exp(xb - m_new), axis=1, keepdims=True)
    m_ref[...] = m_new
    s_ref[...] = s_new
    sx_ref[...] = sx_old + bsum
    xt_ref[...] = xt_old + xt_part

    @pl.when(j == nc - 1)
    def _():
        lse = m_new + jnp.log(s_new)
        row_loss = _C0 + lse - _EPS * sx_ref[...] - _CS * xt_ref[...]
        valid = t != _PAD
        contrib = jnp.sum(jnp.where(valid, row_loss, 0.0)) * inv_denom
        out_ref[...] = jnp.zeros((1, 1, 1), jnp.float32) + contrib


def kernel(x, target):
    batch = x.shape[0]
    n = x.shape[0] * x.shape[1]
    xf = x.reshape(n, _V)
    t = target.reshape(n, 1).astype(jnp.int32)
    nr = n // _R
    nc = _V // _C
    out = pl.pallas_call(
        functools.partial(_body, nc, 1.0 / batch),
        grid=(nr, nc),
        in_specs=[
            pl.BlockSpec((_R, _C), lambda i, j: (i, j)),
            pl.BlockSpec((_R, 1), lambda i, j: (i, 0)),
        ],
        out_specs=pl.BlockSpec((1, 1, 1), lambda i, j: (i, 0, 0)),
        out_shape=jax.ShapeDtypeStruct((nr, 1, 1), jnp.float32),
        scratch_shapes=[
            pltpu.VMEM((_R, 1), jnp.float32),
            pltpu.VMEM((_R, 1), jnp.float32),
            pltpu.VMEM((_R, 1), jnp.float32),
            pltpu.VMEM((_R, 1), jnp.float32),
        ],
        compiler_params=pltpu.CompilerParams(
            dimension_semantics=("parallel", "arbitrary"),
        ),
    )(xf, t)
    return jnp.sum(out)
